# Initial kernel scaffold; baseline (speedup 1.0000x reference)
import functools

import jax
import jax.numpy as jnp
from jax import lax
from jax.experimental import pallas as pl
from jax.experimental.pallas import tpu as pltpu
from jax.experimental.pallas import tpu_sc as plsc

N = 10000
E = 320000
D = 128

NC = 2
NS = 16
NW = NC * NS
EPW = E // NW
C = 80
CHUNKS = EPW // C
RPT = N // NS
ZB = 125

_mesh = plsc.VectorSubcoreMesh(core_axis_name="c", subcore_axis_name="s")


@jax.jit
def _sc_spmm(x, src3, dst3):
  @functools.partial(
      pl.kernel,
      out_type=jax.ShapeDtypeStruct((NC, N, D), jnp.float32),
      mesh=_mesh,
      scratch_types=[
          pltpu.VMEM((CHUNKS, C), jnp.int32),
          pltpu.VMEM((CHUNKS, C), jnp.int32),
          pltpu.VMEM((C, D), jnp.float32),
          pltpu.VMEM((ZB, D), jnp.float32),
          pltpu.VMEM_SHARED((N, D), jnp.float32),
      ],
  )
  def sc_kernel(x_hbm, src_hbm, dst_hbm, out_hbm, src_v, dst_v, rows_v,
                zb_v, t_sh):
    c = lax.axis_index("c")
    s = lax.axis_index("s")
    wid = s * NC + c

    pltpu.sync_copy(src_hbm.at[wid], src_v)
    pltpu.sync_copy(dst_hbm.at[wid], dst_v)

    @pl.loop(0, ZB)
    def _(i):
      @pl.loop(0, D, step=16)
      def _(j):
        zb_v[i, pl.ds(j, 16)] = jnp.zeros((16,), jnp.float32)

    @pl.loop(0, RPT // ZB)
    def _(k):
      pltpu.sync_copy(zb_v, t_sh.at[pl.ds(s * RPT + k * ZB, ZB)])

    plsc.subcore_barrier()

    @pl.loop(0, CHUNKS)
    def _(i):
      pltpu.sync_copy(x_hbm.at[src_v.at[i]], rows_v)
      pltpu.sync_copy(rows_v, t_sh.at[dst_v.at[i]], add=True)

    plsc.subcore_barrier()

    @pl.loop(0, RPT // ZB)
    def _(k):
      pltpu.sync_copy(t_sh.at[pl.ds(s * RPT + k * ZB, ZB)], zb_v)
      pltpu.sync_copy(zb_v, out_hbm.at[c, pl.ds(s * RPT + k * ZB, ZB)])

  return sc_kernel(x, src3, dst3)


def _mm_body(p_ref, w_ref, z_ref):
  t = p_ref[0] + p_ref[1]
  z_ref[...] = jnp.dot(t, w_ref[...], preferred_element_type=jnp.float32)


@jax.jit
def _mm(parts, weight):
  return pl.pallas_call(
      _mm_body,
      grid=(10,),
      in_specs=[
          pl.BlockSpec((2, N // 10, D), lambda i: (0, i, 0)),
          pl.BlockSpec((D, D), lambda i: (0, 0)),
      ],
      out_specs=pl.BlockSpec((N // 10, D), lambda i: (i, 0)),
      out_shape=jax.ShapeDtypeStruct((N, D), jnp.float32),
  )(parts, weight)


def kernel(x, adj, weight):
  src3 = adj[0].reshape(NW, CHUNKS, C)
  dst3 = adj[1].reshape(NW, CHUNKS, C)
  parts = _sc_spmm(x, src3, dst3)
  return _mm(parts, weight)


# speedup vs baseline: 7.7064x; 7.7064x over previous
import functools

import jax
import jax.numpy as jnp
from jax import lax
from jax.experimental import pallas as pl
from jax.experimental.pallas import tpu as pltpu
from jax.experimental.pallas import tpu_sc as plsc

N = 10000
E = 320000
D = 128

NC = 2
NS = 16
NW = NC * NS
EPW = E // NW
C = 80
CHUNKS = EPW // C
NP = 10240
RPT = NP // NS
ZB = 80

_mesh = plsc.VectorSubcoreMesh(core_axis_name="c", subcore_axis_name="s")


@jax.jit
def _sc_spmm(x, src3, dst3):
  @functools.partial(
      pl.kernel,
      out_type=jax.ShapeDtypeStruct((NC, NP, D), jnp.float32),
      mesh=_mesh,
      scratch_types=[
          pltpu.VMEM((CHUNKS, C), jnp.int32),
          pltpu.VMEM((CHUNKS, C), jnp.int32),
          pltpu.VMEM((C, D), jnp.float32),
          pltpu.VMEM_SHARED((NP, D), jnp.float32),
      ],
  )
  def sc_kernel(x_hbm, src_hbm, dst_hbm, out_hbm, src_v, dst_v, rows_v,
                t_sh):
    c = lax.axis_index("c")
    s = lax.axis_index("s")
    wid = s * NC + c

    pltpu.sync_copy(src_hbm.at[wid], src_v)
    pltpu.sync_copy(dst_hbm.at[wid], dst_v)

    @pl.loop(0, ZB)
    def _(i):
      @pl.loop(0, D, step=16)
      def _(j):
        rows_v[i, pl.ds(j, 16)] = jnp.zeros((16,), jnp.float32)

    @pl.loop(0, RPT // ZB)
    def _(k):
      pltpu.sync_copy(rows_v, t_sh.at[pl.ds(s * RPT + k * ZB, ZB)])

    plsc.subcore_barrier()

    @pl.loop(0, CHUNKS)
    def _(i):
      pltpu.sync_copy(x_hbm.at[src_v.at[i]], rows_v)
      pltpu.sync_copy(rows_v, t_sh.at[dst_v.at[i]], add=True)

    plsc.subcore_barrier()

    @pl.loop(0, RPT // ZB)
    def _(k):
      pltpu.sync_copy(t_sh.at[pl.ds(s * RPT + k * ZB, ZB)], rows_v)
      pltpu.sync_copy(rows_v, out_hbm.at[c, pl.ds(s * RPT + k * ZB, ZB)])

  return sc_kernel(x, src3, dst3)


def _mm_body(p_ref, w_ref, z_ref):
  t = p_ref[0] + p_ref[1]
  z_ref[...] = jnp.dot(t, w_ref[...], preferred_element_type=jnp.float32)


@jax.jit
def _mm(parts, weight):
  return pl.pallas_call(
      _mm_body,
      grid=(10,),
      in_specs=[
          pl.BlockSpec((2, N // 10, D), lambda i: (0, i, 0)),
          pl.BlockSpec((D, D), lambda i: (0, 0)),
      ],
      out_specs=pl.BlockSpec((N // 10, D), lambda i: (i, 0)),
      out_shape=jax.ShapeDtypeStruct((N, D), jnp.float32),
  )(parts, weight)


def kernel(x, adj, weight):
  src3 = adj[0].reshape(NW, CHUNKS, C)
  dst3 = adj[1].reshape(NW, CHUNKS, C)
  parts = _sc_spmm(x, src3, dst3)
  return _mm(parts, weight)


# speedup vs baseline: 10.2757x; 1.3334x over previous
import functools

import jax
import jax.numpy as jnp
from jax import lax
from jax.experimental import pallas as pl
from jax.experimental.pallas import tpu as pltpu
from jax.experimental.pallas import tpu_sc as plsc

N = 10000
E = 320000
D = 128

NC = 2
NS = 16
NW = NC * NS
C = 80
CHUNKS = 125
NP = 10240
RPT = NP // NS
ZB = 80
NBUF = 2
ISL = 4

_mesh = plsc.VectorSubcoreMesh(core_axis_name="c", subcore_axis_name="s")


@jax.jit
def _sc_spmm(x, adj4):
  @functools.partial(
      pl.kernel,
      out_type=jax.ShapeDtypeStruct((NC, NP, D), jnp.float32),
      mesh=_mesh,
      scratch_types=[
          pltpu.VMEM((ISL, 2, C), jnp.int32),
          pltpu.VMEM((NBUF, C, D), jnp.float32),
          pltpu.VMEM_SHARED((NP, D), jnp.float32),
          pltpu.SemaphoreType.DMA((NBUF,)),
          pltpu.SemaphoreType.DMA((NBUF,)),
          pltpu.SemaphoreType.DMA((ISL,)),
      ],
  )
  def sc_kernel(x_hbm, adj_hbm, out_hbm, idx_v, rows_v, t_sh, gsem, ssem,
                isem):
    c = lax.axis_index("c")
    s = lax.axis_index("s")
    wid = s * NC + c

    def stage(k, sl):
      return pltpu.make_async_copy(adj_hbm.at[wid, k], idx_v.at[sl],
                                   isem.at[sl])

    def gather(k, b, sl):
      return pltpu.make_async_copy(x_hbm.at[idx_v.at[sl, 0]], rows_v.at[b],
                                   gsem.at[b])

    def scatter(k, b, sl):
      return pltpu.make_async_copy(rows_v.at[b], t_sh.at[idx_v.at[sl, 1]],
                                   ssem.at[b])

    stage(0, 0).start()
    stage(1, 1).start()

    @pl.loop(0, ZB)
    def _(i):
      @pl.loop(0, D, step=16)
      def _(j):
        rows_v[0, i, pl.ds(j, 16)] = jnp.zeros((16,), jnp.float32)

    @pl.loop(0, RPT // ZB)
    def _(k):
      pltpu.sync_copy(rows_v.at[0], t_sh.at[pl.ds(s * RPT + k * ZB, ZB)])

    plsc.subcore_barrier()

    stage(0, 0).wait()
    gather(0, 0, 0).start()

    @pl.loop(0, CHUNKS - 1, step=ISL)
    def _(i):
      for b in range(ISL):
        k = i + b
        rb = b % NBUF
        pb = (b - 1) % NBUF
        sl = b
        sl2 = (b + 2) % ISL
        sl1 = (b + 1) % ISL

        gather(k, rb, sl).wait()

        @pl.when(k > 0)
        def _():
          scatter(k - 1, pb, (b - 1) % ISL).wait()

        scatter(k, rb, sl).start(add=True)

        @pl.when(k + 2 <= CHUNKS - 1)
        def _():
          stage(k + 2, sl2).start()

        @pl.when(k + 1 <= CHUNKS - 1)
        def _():
          stage(k + 1, sl1).wait()
          gather(k + 1, (b + 1) % NBUF, sl1).start()

    gather(CHUNKS - 1, 0, 0).wait()
    scatter(CHUNKS - 2, 1, 3).wait()
    scatter(CHUNKS - 1, 0, 0).start(add=True)
    scatter(CHUNKS - 1, 0, 0).wait()

    plsc.subcore_barrier()

    NWB = RPT // ZB

    def wb_load(k):
      return pltpu.make_async_copy(t_sh.at[pl.ds(s * RPT + k * ZB, ZB)],
                                   rows_v.at[k % NBUF], gsem.at[k % NBUF])

    def wb_store(k):
      return pltpu.make_async_copy(rows_v.at[k % NBUF],
                                   out_hbm.at[c, pl.ds(s * RPT + k * ZB, ZB)],
                                   ssem.at[k % NBUF])

    wb_load(0).start()
    for k in range(NWB):
      wb_load(k).wait()
      wb_store(k).start()
      if k > 0:
        wb_store(k - 1).wait()
      if k + 1 < NWB:
        wb_load(k + 1).start()
    wb_store(NWB - 1).wait()

  return sc_kernel(x, adj4)


def _mm_body(p_ref, w_ref, z_ref):
  t = p_ref[0] + p_ref[1]
  z_ref[...] = jnp.dot(t, w_ref[...], preferred_element_type=jnp.float32)


@jax.jit
def _mm(parts, weight):
  return pl.pallas_call(
      _mm_body,
      grid=(10,),
      in_specs=[
          pl.BlockSpec((2, N // 10, D), lambda i: (0, i, 0)),
          pl.BlockSpec((D, D), lambda i: (0, 0)),
      ],
      out_specs=pl.BlockSpec((N // 10, D), lambda i: (i, 0)),
      out_shape=jax.ShapeDtypeStruct((N, D), jnp.float32),
  )(parts, weight)


def kernel(x, adj, weight):
  adj4 = adj.reshape(2, NW, CHUNKS, C).transpose(1, 2, 0, 3)
  parts = _sc_spmm(x, adj4)
  return _mm(parts, weight)
